# Initial kernel scaffold; baseline (speedup 1.0000x reference)
import jax
import jax.numpy as jnp
from jax.experimental import pallas as pl


def kernel(inputs, indices):
    raise NotImplementedError("write your pallas kernel here")


# speedup vs baseline: 3.4729x; 3.4729x over previous
import jax
import jax.numpy as jnp
from jax import lax
from jax.experimental import pallas as pl
from jax.experimental.pallas import tpu as pltpu
from jax.experimental.pallas import tpu_sc as plsc

B, C, H, W = 2, 768, 112, 112
HW = H * W
PLANE = 4 * HW
ROWS = B * C
NW = 32
ROWS_PER_W = ROWS // NW


def _unpool_body(vals_hbm, idx_hbm, out_hbm, idx_v, val_v, plane_v):
    wid = lax.axis_index("s") * 2 + lax.axis_index("c")

    zeros16 = jnp.zeros((16,), jnp.float32)

    def do_plane(p, carry):
        row = wid * ROWS_PER_W + p
        pltpu.sync_copy(idx_hbm.at[pl.ds(row * HW, HW)], idx_v)
        pltpu.sync_copy(vals_hbm.at[pl.ds(row * HW, HW)], val_v)

        base = jnp.full((16,), row * PLANE, jnp.int32)

        def zbody(i, c):
            plane_v[pl.ds(i * 16, 16)] = zeros16
            return c

        lax.fori_loop(0, PLANE // 16, zbody, 0)

        def sbody(i, c):
            ii = idx_v[pl.ds(i * 16, 16)] - base
            vv = val_v[pl.ds(i * 16, 16)]
            _, keep = plsc.scan_count(ii)
            plsc.store_scatter(plane_v, [ii], vv, mask=keep)
            return c

        lax.fori_loop(0, HW // 16, sbody, 0)

        pltpu.sync_copy(plane_v, out_hbm.at[pl.ds(row * PLANE, PLANE)])
        return carry

    lax.fori_loop(0, ROWS_PER_W, do_plane, 0)


@jax.jit
def kernel(inputs, indices):
    vals = inputs.reshape(ROWS * HW)
    rows = jnp.arange(ROWS, dtype=jnp.int32)[:, None]
    keys = (indices.reshape(ROWS, HW) + rows * PLANE).reshape(ROWS * HW)
    skey, sval = lax.sort((keys, vals), num_keys=1, is_stable=False)

    mesh = plsc.VectorSubcoreMesh(core_axis_name="c", subcore_axis_name="s")
    out = pl.kernel(
        _unpool_body,
        mesh=mesh,
        compiler_params=pltpu.CompilerParams(
            needs_layout_passes=False,
            use_tc_tiling_on_sc=False,
        ),
        out_type=jax.ShapeDtypeStruct((ROWS * PLANE,), jnp.float32),
        scratch_types=[
            pltpu.VMEM((HW,), jnp.int32),
            pltpu.VMEM((HW,), jnp.float32),
            pltpu.VMEM((PLANE,), jnp.float32),
        ],
    )(sval, skey)
    return out.reshape(B, C, 2 * H, 2 * W)


# speedup vs baseline: 3.5690x; 1.0277x over previous
import jax
import jax.numpy as jnp
from jax import lax
from jax.experimental import pallas as pl
from jax.experimental.pallas import tpu as pltpu
from jax.experimental.pallas import tpu_sc as plsc

B, C, H, W = 2, 768, 112, 112
HW = H * W
PLANE = 4 * HW
ROWS = B * C
NW = 32
ROWS_PER_W = ROWS // NW


_UNROLL = 16


def _unpool_body(vals_hbm, idx_hbm, out_hbm, idx_v, val_v, plane_v):
    wid = lax.axis_index("s") * 2 + lax.axis_index("c")

    zeros16 = jnp.zeros((16,), jnp.float32)

    def zbody(i, c):
        for u in range(_UNROLL):
            plane_v[pl.ds((i * _UNROLL + u) * 16, 16)] = zeros16
        return c

    lax.fori_loop(0, PLANE // 16 // _UNROLL, zbody, 0)

    sentinel = jnp.full((16,), -1, jnp.int32)

    def do_plane(p, carry):
        row = wid * ROWS_PER_W + p
        pltpu.sync_copy(idx_hbm.at[pl.ds(row * HW, HW)], idx_v.at[pl.ds(0, HW)])
        pltpu.sync_copy(vals_hbm.at[pl.ds(row * HW, HW)], val_v)
        idx_v[pl.ds(HW, 16)] = sentinel

        base = jnp.full((16,), row * PLANE, jnp.int32)

        def sbody(i, c):
            for u in range(_UNROLL):
                off = (i * _UNROLL + u) * 16
                ii = idx_v[pl.ds(off, 16)]
                keep = ii != idx_v[pl.ds(off + 1, 16)]
                vv = val_v[pl.ds(off, 16)]
                plsc.store_scatter(plane_v, [ii - base], vv, mask=keep)
            return c

        lax.fori_loop(0, HW // 16 // _UNROLL, sbody, 0)

        pltpu.sync_copy(plane_v, out_hbm.at[pl.ds(row * PLANE, PLANE)])

        def wipe_body(i, c):
            for u in range(_UNROLL):
                off = (i * _UNROLL + u) * 16
                plsc.store_scatter(plane_v, [idx_v[pl.ds(off, 16)] - base], zeros16)
            return c

        lax.fori_loop(0, HW // 16 // _UNROLL, wipe_body, 0)
        return carry

    lax.fori_loop(0, ROWS_PER_W, do_plane, 0)


@jax.jit
def kernel(inputs, indices):
    vals = inputs.reshape(ROWS * HW)
    rows = jnp.arange(ROWS, dtype=jnp.int32)[:, None]
    keys = (indices.reshape(ROWS, HW) + rows * PLANE).reshape(ROWS * HW)
    skey, sval = lax.sort((keys, vals), num_keys=1, is_stable=False)

    mesh = plsc.VectorSubcoreMesh(core_axis_name="c", subcore_axis_name="s")
    out = pl.kernel(
        _unpool_body,
        mesh=mesh,
        compiler_params=pltpu.CompilerParams(
            needs_layout_passes=False,
            use_tc_tiling_on_sc=False,
        ),
        out_type=jax.ShapeDtypeStruct((ROWS * PLANE,), jnp.float32),
        scratch_types=[
            pltpu.VMEM((HW + 16,), jnp.int32),
            pltpu.VMEM((HW,), jnp.float32),
            pltpu.VMEM((PLANE,), jnp.float32),
        ],
    )(sval, skey)
    return out.reshape(B, C, 2 * H, 2 * W)


# speedup vs baseline: 3.5743x; 1.0015x over previous
import jax
import jax.numpy as jnp
from jax import lax
from jax.experimental import pallas as pl
from jax.experimental.pallas import tpu as pltpu
from jax.experimental.pallas import tpu_sc as plsc

B, C, H, W = 2, 768, 112, 112
HW = H * W
PLANE = 4 * HW
ROWS = B * C
NW = 32
ROWS_PER_W = ROWS // NW


_UNROLL = 16


def _unpool_body(
    vals_hbm, idx_hbm, out_hbm, idx0, val0, idx1, val1, plane_v, sem0, sem1
):
    wid = lax.axis_index("s") * 2 + lax.axis_index("c")

    zeros16 = jnp.zeros((16,), jnp.float32)
    sentinel = jnp.full((16,), -1, jnp.int32)

    def zbody(i, c):
        for u in range(_UNROLL):
            plane_v[pl.ds((i * _UNROLL + u) * 16, 16)] = zeros16
        return c

    lax.fori_loop(0, PLANE // 16 // _UNROLL, zbody, 0)

    def process(row, idx_v, val_v):
        idx_v[pl.ds(HW, 16)] = sentinel
        base = jnp.full((16,), row * PLANE, jnp.int32)

        def sbody(i, c):
            for u in range(_UNROLL):
                off = (i * _UNROLL + u) * 16
                ii = idx_v[pl.ds(off, 16)]
                keep = ii != idx_v[pl.ds(off + 1, 16)]
                vv = val_v[pl.ds(off, 16)]
                plsc.store_scatter(plane_v, [ii - base], vv, mask=keep)
            return c

        lax.fori_loop(0, HW // 16 // _UNROLL, sbody, 0)

        pltpu.sync_copy(plane_v, out_hbm.at[pl.ds(row * PLANE, PLANE)])

        def wipe_body(i, c):
            for u in range(_UNROLL):
                off = (i * _UNROLL + u) * 16
                plsc.store_scatter(plane_v, [idx_v[pl.ds(off, 16)] - base], zeros16)
            return c

        lax.fori_loop(0, HW // 16 // _UNROLL, wipe_body, 0)

    def do_pair(i, carry):
        r0 = wid * ROWS_PER_W + 2 * i
        r1 = r0 + 1
        c0a = pltpu.async_copy(
            idx_hbm.at[pl.ds(r0 * HW, HW)], idx0.at[pl.ds(0, HW)], sem0
        )
        c0b = pltpu.async_copy(vals_hbm.at[pl.ds(r0 * HW, HW)], val0, sem0)
        c1a = pltpu.async_copy(
            idx_hbm.at[pl.ds(r1 * HW, HW)], idx1.at[pl.ds(0, HW)], sem1
        )
        c1b = pltpu.async_copy(vals_hbm.at[pl.ds(r1 * HW, HW)], val1, sem1)
        c0a.wait()
        c0b.wait()
        process(r0, idx0, val0)
        c1a.wait()
        c1b.wait()
        process(r1, idx1, val1)
        return carry

    lax.fori_loop(0, ROWS_PER_W // 2, do_pair, 0)


@jax.jit
def kernel(inputs, indices):
    vals = inputs.reshape(ROWS * HW)
    rows = jnp.arange(ROWS, dtype=jnp.int32)[:, None]
    keys = (indices.reshape(ROWS, HW) + rows * PLANE).reshape(ROWS * HW)
    skey, sval = lax.sort((keys, vals), num_keys=1, is_stable=False)

    mesh = plsc.VectorSubcoreMesh(core_axis_name="c", subcore_axis_name="s")
    out = pl.kernel(
        _unpool_body,
        mesh=mesh,
        compiler_params=pltpu.CompilerParams(
            needs_layout_passes=False,
            use_tc_tiling_on_sc=False,
        ),
        out_type=jax.ShapeDtypeStruct((ROWS * PLANE,), jnp.float32),
        scratch_types=[
            pltpu.VMEM((HW + 16,), jnp.int32),
            pltpu.VMEM((HW,), jnp.float32),
            pltpu.VMEM((HW + 16,), jnp.int32),
            pltpu.VMEM((HW,), jnp.float32),
            pltpu.VMEM((PLANE,), jnp.float32),
            pltpu.SemaphoreType.DMA,
            pltpu.SemaphoreType.DMA,
        ],
    )(sval, skey)
    return out.reshape(B, C, 2 * H, 2 * W)
